# Initial kernel scaffold; baseline (speedup 1.0000x reference)
import jax
import jax.numpy as jnp
from jax.experimental import pallas as pl


def kernel(x, edge_index, edge_weight, W, ew_param):
    raise NotImplementedError("write your pallas kernel here")


# speedup vs baseline: 3.2313x; 3.2313x over previous
import functools

import jax
import jax.numpy as jnp
from jax import lax
from jax.experimental import pallas as pl
from jax.experimental.pallas import tpu as pltpu
from jax.experimental.pallas import tpu_sc as plsc

N_NODES = 10000
N_EDGES = 160000
IN_CH = 256
OUT_CH = 256

_NC = 2
_NS = 16
_L = 16
_HALF = OUT_CH // 2
_CH = 80
_CHUNKS_PER_TILE = N_EDGES // (_NS * _CH)
_RPT_BIG = 640
_RPT_LAST = N_NODES - (_NS - 1) * _RPT_BIG
_ZROWS = 80


def _mm_body(p_ref, x_ref, w_ref, o_ref):
    p = p_ref[0]
    acc = lax.dot_general(x_ref[...], w_ref[0],
                          (((1,), (1,)), ((), ())),
                          preferred_element_type=jnp.float32)
    o_ref[...] = (acc * p)[None]


def _project(ew_param, x, w2):
    bm = 1000
    return pl.pallas_call(
        _mm_body,
        grid=(2, N_NODES // bm),
        in_specs=[
            pl.BlockSpec(memory_space=pltpu.SMEM),
            pl.BlockSpec((bm, IN_CH), lambda h, i: (i, 0)),
            pl.BlockSpec((1, _HALF, IN_CH), lambda h, i: (h, 0, 0)),
        ],
        out_specs=pl.BlockSpec((1, bm, _HALF), lambda h, i: (h, i, 0)),
        out_shape=jax.ShapeDtypeStruct((2, N_NODES, _HALF), jnp.float32),
    )(ew_param, x, w2)


def _agg_body(xls_hbm, ei_hbm, ew_hbm, out_hbm,
              acc, src_b, dst_b, ew_b, rows, sem):
    c = lax.axis_index("c")
    s = lax.axis_index("s")

    def _zero_rows(i, carry):
        rows[i // 8, pl.ds((i % 8) * _L, _L)] = jnp.zeros((_L,), jnp.float32)
        return carry
    lax.fori_loop(0, _ZROWS * 8, _zero_rows, 0)

    r0 = s * _RPT_BIG
    nz = jnp.where(s < _NS - 1, _RPT_BIG // _ZROWS, _RPT_LAST // _ZROWS)

    def _zero_acc(k, carry):
        pltpu.sync_copy(rows, acc.at[pl.ds(r0 + k * _ZROWS, _ZROWS)])
        return carry
    lax.fori_loop(0, nz, _zero_acc, 0)
    plsc.subcore_barrier()

    pltpu.sync_copy(ei_hbm.at[1, s], dst_b)

    def _chunk(j, carry):
        pltpu.sync_copy(ei_hbm.at[0, s, j], src_b.at[0])
        pltpu.sync_copy(ew_hbm.at[s, j], ew_b.at[0])
        pltpu.async_copy(xls_hbm.at[c].at[src_b.at[0]], rows, sem).wait()

        def _scale(g, carry2):
            evw = ew_b[0, pl.ds(g * _L, _L)]
            base = g * _L
            for r16 in range(_L):
                sv = evw[r16]
                for k in range(_HALF // _L):
                    sl = pl.ds(k * _L, _L)
                    rows[base + r16, sl] = rows[base + r16, sl] * sv
            return carry2
        lax.fori_loop(0, _CH // _L, _scale, 0)

        pltpu.sync_copy(rows, acc.at[dst_b.at[j]], add=True)
        return carry
    lax.fori_loop(0, _CHUNKS_PER_TILE, _chunk, 0)
    plsc.subcore_barrier()

    for ci, c0 in ((0, 0), (1, _HALF)):
        @pl.when((c == ci) & (s < _NS - 1))
        def _(c0=c0):
            pltpu.sync_copy(acc.at[pl.ds(r0, _RPT_BIG)],
                            out_hbm.at[pl.ds(r0, _RPT_BIG), pl.ds(c0, _HALF)])

        @pl.when((c == ci) & (s == _NS - 1))
        def _(c0=c0):
            pltpu.sync_copy(acc.at[pl.ds(r0, _RPT_LAST)],
                            out_hbm.at[pl.ds(r0, _RPT_LAST), pl.ds(c0, _HALF)])


def _aggregate(xls, ei, ew):
    mesh = plsc.VectorSubcoreMesh(core_axis_name="c", subcore_axis_name="s")
    return pl.kernel(
        _agg_body,
        out_type=jax.ShapeDtypeStruct((N_NODES, OUT_CH), jnp.float32),
        mesh=mesh,
        scratch_types=[
            pltpu.VMEM_SHARED((N_NODES, _HALF), jnp.float32),
            pltpu.VMEM((1, _CH), jnp.int32),
            pltpu.VMEM((_CHUNKS_PER_TILE, _CH), jnp.int32),
            pltpu.VMEM((1, _CH), jnp.float32),
            pltpu.VMEM((_CH, _HALF), jnp.float32),
            pltpu.SemaphoreType.DMA,
        ],
    )(xls, ei, ew)


def kernel(x, edge_index, edge_weight, W, ew_param):
    w2 = W.reshape(2, _HALF, IN_CH)
    xls = _project(ew_param, x, w2)
    ei = edge_index.astype(jnp.int32).reshape(2, _NS, _CHUNKS_PER_TILE, _CH)
    ew = edge_weight.reshape(_NS, _CHUNKS_PER_TILE, _CH)
    return _aggregate(xls, ei, ew)
